# Initial kernel scaffold; baseline (speedup 1.0000x reference)
import jax
import jax.numpy as jnp
from jax.experimental import pallas as pl


def kernel(features, pe_features, W_clf, b_clf, W_r1, W_r2, W_r3, weight, nodes, labels, neigh_r1, neigh_r2, neigh_r3, train_pos):
    raise NotImplementedError("write your pallas kernel here")


# speedup vs baseline: 3.0444x; 3.0444x over previous
import functools

import jax
import jax.numpy as jnp
from jax import lax
from jax.experimental import pallas as pl
from jax.experimental.pallas import tpu as pltpu
from jax.experimental.pallas import tpu_sc as plsc

_N, _FEAT, _EMB, _B, _DEG = 100000, 128, 64, 8192, 16
_PBLK = 1024
_NPAD = 98 * _PBLK
_NW = 32
_RPW = (3 * _B) // _NW
_CH = 64
_SPW = _B // _NW
_K = 8


def _p_body(x_ref, w_ref, b_ref, o_ref):
    s = lax.dot_general(x_ref[...], w_ref[...], (((1,), (0,)), ((), ())),
                        preferred_element_type=jnp.float32) + b_ref[...]
    m = jnp.max(s, axis=1, keepdims=True)
    e = jnp.exp(s - m)
    p2 = e / jnp.sum(e, axis=1, keepdims=True)
    col1 = lax.broadcasted_iota(jnp.int32, (1, 2), 1) == 1
    o_ref[...] = jnp.sum(jnp.where(col1, p2, 0.0), axis=1)


def _compute_p(features, W_clf, b_clf):
    grid = _NPAD // _PBLK
    return pl.pallas_call(
        _p_body,
        grid=(grid,),
        in_specs=[
            pl.BlockSpec((_PBLK, _FEAT), lambda i: (i, 0)),
            pl.BlockSpec((_FEAT, 2), lambda i: (0, 0)),
            pl.BlockSpec((1, 2), lambda i: (0, 0)),
        ],
        out_specs=pl.BlockSpec((_PBLK,), lambda i: (i,)),
        out_shape=jax.ShapeDtypeStruct((_NPAD,), jnp.float32),
    )(features, W_clf, b_clf.reshape(1, 2))


def _sc_body(feat_hbm, p_hbm, nodes_hbm, ncat_hbm, self_hbm, agg_hbm,
             p_v, nodes_v, neigh_v, sel_v, rows_v, agg_v, sem):
    wid = lax.axis_index("s") * 2 + lax.axis_index("c")
    pltpu.sync_copy(p_hbm, p_v)

    sbase = wid * _SPW
    for j in range(_SPW // _CH):
        pltpu.sync_copy(nodes_hbm.at[pl.ds(sbase + j * _CH, _CH)], nodes_v)
        pltpu.async_copy(feat_hbm.at[nodes_v], rows_v, sem).wait()
        pltpu.sync_copy(rows_v, self_hbm.at[pl.ds(sbase + j * _CH, _CH), :])

    base = wid * _RPW
    selmask = lax.iota(jnp.int32, 16) < _K

    def chunk_body(ci, carry):
        cbase = base + ci * _CH
        pltpu.sync_copy(nodes_hbm.at[pl.ds(lax.rem(cbase, _B), _CH)], nodes_v)
        pltpu.sync_copy(ncat_hbm.at[pl.ds(cbase, _CH), :], neigh_v)

        def sel_body(b, c2):
            ids = neigh_v[b, :]
            pn = plsc.load_gather(p_v, [ids])
            bvec = jnp.full((16,), b, dtype=jnp.int32)
            nvec = plsc.load_gather(nodes_v, [bvec])
            pc = plsc.load_gather(p_v, [nvec])
            dist = jnp.abs(pn - pc)
            _, sids = plsc.sort_key_val(dist, ids)
            plsc.store_compressed(sel_v.at[pl.ds(b * _K, 16)], sids,
                                  mask=selmask)
            return c2

        lax.fori_loop(0, _CH, sel_body, 0)

        def g_body(g, c2):
            idx = sel_v.at[pl.ds(g * 64, 64)]
            pltpu.async_copy(feat_hbm.at[idx], rows_v, sem).wait()

            def j_body(j, c3):
                def c_body(c, c4):
                    acc = rows_v[j * _K, pl.ds(c * 16, 16)]
                    for k in range(1, _K):
                        acc = acc + rows_v[j * _K + k, pl.ds(c * 16, 16)]
                    agg_v[j, pl.ds(c * 16, 16)] = acc
                    return c4

                return lax.fori_loop(0, _FEAT // 16, c_body, c3)

            lax.fori_loop(0, 64 // _K, j_body, 0)
            pltpu.sync_copy(agg_v, agg_hbm.at[pl.ds(cbase + g * 8, 8), :])
            return c2

        lax.fori_loop(0, _CH * _K // 64, g_body, 0)
        return carry

    lax.fori_loop(0, _RPW // _CH, chunk_body, 0)


def _sc_gather(features, p_all, nodes, ncat):
    mesh = plsc.VectorSubcoreMesh(core_axis_name="c", subcore_axis_name="s")
    call = pl.kernel(
        _sc_body,
        out_type=(
            jax.ShapeDtypeStruct((_B, _FEAT), jnp.float32),
            jax.ShapeDtypeStruct((3 * _B, _FEAT), jnp.float32),
        ),
        mesh=mesh,
        compiler_params=pltpu.CompilerParams(needs_layout_passes=False),
        scratch_types=[
            pltpu.VMEM((_NPAD,), jnp.float32),
            pltpu.VMEM((_CH,), jnp.int32),
            pltpu.VMEM((_CH, _DEG), jnp.int32),
            pltpu.VMEM((_CH * _K + 16, ), jnp.int32),
            pltpu.VMEM((64, _FEAT), jnp.float32),
            pltpu.VMEM((8, _FEAT), jnp.float32),
            pltpu.SemaphoreType.DMA,
        ],
    )
    return call(features, p_all, nodes, ncat)


def _c_body(self_ref, a1, a2, a3, wr1, wr2, wr3, ws, w1, w2, w3, o_ref):
    acc = lax.dot_general(ws[...], self_ref[...], (((0,), (1,)), ((), ())),
                          preferred_element_type=jnp.float32)
    for a, wr, wc in ((a1, wr1, w1), (a2, wr2, w2), (a3, wr3, w3)):
        h = jnp.maximum(
            jnp.dot(a[...], wr[...], preferred_element_type=jnp.float32), 0.0)
        acc = acc + lax.dot_general(wc[...], h, (((0,), (1,)), ((), ())),
                                    preferred_element_type=jnp.float32)
    o_ref[...] = jnp.maximum(acc, 0.0)


def _combine(self_feats, aggsum, wr1, wr2, wr3, ws, w1, w2, w3):
    bk = 1024
    grid = _B // bk
    wspec = pl.BlockSpec((_FEAT, _EMB), lambda i: (0, 0))
    especk = pl.BlockSpec((_EMB, _EMB), lambda i: (0, 0))
    return pl.pallas_call(
        _c_body,
        grid=(grid,),
        in_specs=[
            pl.BlockSpec((bk, _FEAT), lambda i: (i, 0)),
            pl.BlockSpec((bk, _FEAT), lambda i: (i, 0)),
            pl.BlockSpec((bk, _FEAT), lambda i: (i + grid, 0)),
            pl.BlockSpec((bk, _FEAT), lambda i: (i + 2 * grid, 0)),
            wspec, wspec, wspec, wspec,
            especk, especk, especk,
        ],
        out_specs=pl.BlockSpec((_EMB, bk), lambda i: (0, i)),
        out_shape=jax.ShapeDtypeStruct((_EMB, _B), jnp.float32),
    )(self_feats, aggsum, aggsum, aggsum, wr1, wr2, wr3, ws, w1, w2, w3)


def kernel(features, pe_features, W_clf, b_clf, W_r1, W_r2, W_r3, weight,
           nodes, labels, neigh_r1, neigh_r2, neigh_r3, train_pos):
    p_all = _compute_p(features, W_clf, b_clf)
    ncat = jnp.concatenate([neigh_r1, neigh_r2, neigh_r3], axis=0)
    self_feats, aggsum = _sc_gather(features, p_all, nodes, ncat)
    scale = jnp.float32(1.0 / _K)
    out = _combine(
        self_feats, aggsum,
        W_r1 * scale, W_r2 * scale, W_r3 * scale,
        weight[0:_FEAT],
        weight[_FEAT:_FEAT + _EMB],
        weight[_FEAT + _EMB:_FEAT + 2 * _EMB],
        weight[_FEAT + 2 * _EMB:_FEAT + 3 * _EMB],
    )
    return out


# speedup vs baseline: 4.2060x; 1.3815x over previous
import functools

import jax
import jax.numpy as jnp
from jax import lax
from jax.experimental import pallas as pl
from jax.experimental.pallas import tpu as pltpu
from jax.experimental.pallas import tpu_sc as plsc

_N, _FEAT, _EMB, _B, _DEG = 100000, 128, 64, 8192, 16
_PBLK = 1024
_NPAD = 98 * _PBLK
_NW = 32
_RPW = (3 * _B) // _NW
_CH = 64
_SPW = _B // _NW
_K = 8


def _p_body(x_ref, w_ref, b_ref, o_ref):
    s = lax.dot_general(x_ref[...], w_ref[...], (((1,), (0,)), ((), ())),
                        preferred_element_type=jnp.float32) + b_ref[...]
    m = jnp.max(s, axis=1, keepdims=True)
    e = jnp.exp(s - m)
    p2 = e / jnp.sum(e, axis=1, keepdims=True)
    col1 = lax.broadcasted_iota(jnp.int32, (1, 2), 1) == 1
    o_ref[...] = jnp.sum(jnp.where(col1, p2, 0.0), axis=1)


def _compute_p(features, W_clf, b_clf):
    grid = _NPAD // _PBLK
    return pl.pallas_call(
        _p_body,
        grid=(grid,),
        in_specs=[
            pl.BlockSpec((_PBLK, _FEAT), lambda i: (i, 0)),
            pl.BlockSpec((_FEAT, 2), lambda i: (0, 0)),
            pl.BlockSpec((1, 2), lambda i: (0, 0)),
        ],
        out_specs=pl.BlockSpec((_PBLK,), lambda i: (i,)),
        out_shape=jax.ShapeDtypeStruct((_NPAD,), jnp.float32),
    )(features, W_clf, b_clf.reshape(1, 2))


_GB = 16
_GR = _GB * _K
_NG = _RPW // _GB


def _sc_body(feat_hbm, p_hbm, nodes_hbm, ncatf_hbm, self_hbm, agg_hbm,
             neighf_v, pnf_v, nodes3_v, pc_v, snodes_v, sel_v,
             rows0, rows1, agg0, agg1,
             semb, semg0, semg1, semw0, semw1):
    wid = lax.axis_index("s") * 2 + lax.axis_index("c")
    base = wid * _RPW

    cds = [pltpu.async_copy(
        ncatf_hbm.at[pl.ds(base * _DEG, _RPW * _DEG)], neighf_v, semb)]
    for ci in range(_RPW // _CH):
        cds.append(pltpu.async_copy(
            nodes_hbm.at[pl.ds(lax.rem(base + ci * _CH, _B), _CH)],
            nodes3_v.at[pl.ds(ci * _CH, _CH)], semb))
    cds.append(pltpu.async_copy(
        nodes_hbm.at[pl.ds(wid * _SPW, _SPW)], snodes_v, semb))
    for cd in cds:
        cd.wait()

    for grp in range(0, _RPW * _DEG // 128, 8):
        cds = [pltpu.async_copy(
            p_hbm.at[neighf_v.at[pl.ds((grp + i) * 128, 128)]],
            pnf_v.at[pl.ds((grp + i) * 128, 128)], semb) for i in range(8)]
        for cd in cds:
            cd.wait()
    cds = [pltpu.async_copy(
        p_hbm.at[nodes3_v.at[pl.ds(i * 128, 128)]],
        pc_v.at[pl.ds(i * 128, 128)], semb) for i in range(_RPW // 128)]
    for cd in cds:
        cd.wait()

    selmask = lax.iota(jnp.int32, 16) < _K

    def sel_body(b, c):
        ids = neighf_v[pl.ds(b * _DEG, _DEG)]
        pn = pnf_v[pl.ds(b * _DEG, _DEG)]
        pc = plsc.load_gather(pc_v, [jnp.full((16,), b, dtype=jnp.int32)])
        dist = jnp.abs(pn - pc)
        _, sids = plsc.sort_key_val(dist, ids)
        plsc.store_compressed(sel_v.at[pl.ds(b * _K, 16)], sids, mask=selmask)
        return c

    lax.fori_loop(0, _RPW, sel_body, 0)

    def fire(g, buf, sem):
        gg = lax.rem(g, _NG)
        pltpu.async_copy(feat_hbm.at[sel_v.at[pl.ds(gg * _GR, _GR)]], buf,
                         sem)

    def gwait(buf, sem):
        pltpu.make_async_copy(
            feat_hbm.at[sel_v.at[pl.ds(0, _GR)]], buf, sem).wait()

    def wwait(agg, sem):
        pltpu.make_async_copy(agg, agg_hbm.at[pl.ds(0, _GB), :], sem).wait()

    def reduce_into(rows, agg):
        def j_body(j, c):
            for cc in range(_FEAT // 16):
                acc = rows[j * _K, pl.ds(cc * 16, 16)]
                for k in range(1, _K):
                    acc = acc + rows[j * _K + k, pl.ds(cc * 16, 16)]
                agg[j, pl.ds(cc * 16, 16)] = acc
            return c

        lax.fori_loop(0, _GB, j_body, 0)

    fire(0, rows0, semg0)

    def pipe_body(i, c):
        g0 = 2 * i
        g1 = g0 + 1
        fire(g1, rows1, semg1)
        gwait(rows0, semg0)

        @pl.when(i > 0)
        def _():
            wwait(agg0, semw0)

        reduce_into(rows0, agg0)
        pltpu.async_copy(agg0, agg_hbm.at[pl.ds(base + g0 * _GB, _GB), :],
                         semw0)
        fire(g0 + 2, rows0, semg0)
        gwait(rows1, semg1)

        @pl.when(i > 0)
        def _():
            wwait(agg1, semw1)

        reduce_into(rows1, agg1)
        pltpu.async_copy(agg1, agg_hbm.at[pl.ds(base + g1 * _GB, _GB), :],
                         semw1)
        return c

    lax.fori_loop(0, _NG // 2, pipe_body, 0)
    gwait(rows0, semg0)
    wwait(agg0, semw0)
    wwait(agg1, semw1)

    fire0 = pltpu.async_copy(
        feat_hbm.at[snodes_v.at[pl.ds(0, _GR)]], rows0, semg0)
    fire1 = pltpu.async_copy(
        feat_hbm.at[snodes_v.at[pl.ds(_GR, _GR)]], rows1, semg1)
    fire0.wait()
    pltpu.sync_copy(rows0, self_hbm.at[pl.ds(wid * _SPW, _GR), :])
    fire1.wait()
    pltpu.sync_copy(rows1, self_hbm.at[pl.ds(wid * _SPW + _GR, _GR), :])


def _sc_gather(features, p_all, nodes, ncatf):
    mesh = plsc.VectorSubcoreMesh(core_axis_name="c", subcore_axis_name="s")
    call = pl.kernel(
        _sc_body,
        out_type=(
            jax.ShapeDtypeStruct((_B, _FEAT), jnp.float32),
            jax.ShapeDtypeStruct((3 * _B, _FEAT), jnp.float32),
        ),
        mesh=mesh,
        compiler_params=pltpu.CompilerParams(needs_layout_passes=False),
        scratch_types=[
            pltpu.VMEM((_RPW * _DEG,), jnp.int32),
            pltpu.VMEM((_RPW * _DEG,), jnp.float32),
            pltpu.VMEM((_RPW,), jnp.int32),
            pltpu.VMEM((_RPW,), jnp.float32),
            pltpu.VMEM((_SPW,), jnp.int32),
            pltpu.VMEM((_RPW * _K + 16,), jnp.int32),
            pltpu.VMEM((_GR, _FEAT), jnp.float32),
            pltpu.VMEM((_GR, _FEAT), jnp.float32),
            pltpu.VMEM((_GB, _FEAT), jnp.float32),
            pltpu.VMEM((_GB, _FEAT), jnp.float32),
            pltpu.SemaphoreType.DMA,
            pltpu.SemaphoreType.DMA,
            pltpu.SemaphoreType.DMA,
            pltpu.SemaphoreType.DMA,
            pltpu.SemaphoreType.DMA,
        ],
    )
    return call(features, p_all, nodes, ncatf)


def _c_body(self_ref, a1, a2, a3, wr1, wr2, wr3, ws, w1, w2, w3, o_ref):
    acc = lax.dot_general(ws[...], self_ref[...], (((0,), (1,)), ((), ())),
                          preferred_element_type=jnp.float32)
    for a, wr, wc in ((a1, wr1, w1), (a2, wr2, w2), (a3, wr3, w3)):
        h = jnp.maximum(
            jnp.dot(a[...], wr[...], preferred_element_type=jnp.float32), 0.0)
        acc = acc + lax.dot_general(wc[...], h, (((0,), (1,)), ((), ())),
                                    preferred_element_type=jnp.float32)
    o_ref[...] = jnp.maximum(acc, 0.0)


def _combine(self_feats, aggsum, wr1, wr2, wr3, ws, w1, w2, w3):
    bk = 1024
    grid = _B // bk
    wspec = pl.BlockSpec((_FEAT, _EMB), lambda i: (0, 0))
    especk = pl.BlockSpec((_EMB, _EMB), lambda i: (0, 0))
    return pl.pallas_call(
        _c_body,
        grid=(grid,),
        in_specs=[
            pl.BlockSpec((bk, _FEAT), lambda i: (i, 0)),
            pl.BlockSpec((bk, _FEAT), lambda i: (i, 0)),
            pl.BlockSpec((bk, _FEAT), lambda i: (i + grid, 0)),
            pl.BlockSpec((bk, _FEAT), lambda i: (i + 2 * grid, 0)),
            wspec, wspec, wspec, wspec,
            especk, especk, especk,
        ],
        out_specs=pl.BlockSpec((_EMB, bk), lambda i: (0, i)),
        out_shape=jax.ShapeDtypeStruct((_EMB, _B), jnp.float32),
    )(self_feats, aggsum, aggsum, aggsum, wr1, wr2, wr3, ws, w1, w2, w3)


def kernel(features, pe_features, W_clf, b_clf, W_r1, W_r2, W_r3, weight,
           nodes, labels, neigh_r1, neigh_r2, neigh_r3, train_pos):
    p_all = _compute_p(features, W_clf, b_clf)
    ncatf = jnp.concatenate([neigh_r1, neigh_r2, neigh_r3], axis=0).reshape(-1)
    self_feats, aggsum = _sc_gather(features, p_all, nodes, ncatf)
    scale = jnp.float32(1.0 / _K)
    out = _combine(
        self_feats, aggsum,
        W_r1 * scale, W_r2 * scale, W_r3 * scale,
        weight[0:_FEAT],
        weight[_FEAT:_FEAT + _EMB],
        weight[_FEAT + _EMB:_FEAT + 2 * _EMB],
        weight[_FEAT + 2 * _EMB:_FEAT + 3 * _EMB],
    )
    return out


# speedup vs baseline: 4.9984x; 1.1884x over previous
import functools

import jax
import jax.numpy as jnp
from jax import lax
from jax.experimental import pallas as pl
from jax.experimental.pallas import tpu as pltpu
from jax.experimental.pallas import tpu_sc as plsc

_N, _FEAT, _EMB, _B, _DEG = 100000, 128, 64, 8192, 16
_PBLK = 1024
_NPAD = 98 * _PBLK
_NW = 32
_RPW = (3 * _B) // _NW
_CH = 64
_SPW = _B // _NW
_K = 8


def _p_body(x_ref, w_ref, b_ref, o_ref):
    sT = lax.dot_general(w_ref[...], x_ref[...], (((0,), (1,)), ((), ())),
                         preferred_element_type=jnp.float32) + b_ref[...]
    s0 = sT[0, :]
    s1 = sT[1, :]
    m = jnp.maximum(s0, s1)
    e0 = jnp.exp(s0 - m)
    e1 = jnp.exp(s1 - m)
    o_ref[...] = e1 / (e0 + e1)


def _compute_p(features, W_clf, b_clf):
    grid = _NPAD // _PBLK
    return pl.pallas_call(
        _p_body,
        grid=(grid,),
        in_specs=[
            pl.BlockSpec((_PBLK, _FEAT), lambda i: (i, 0)),
            pl.BlockSpec((_FEAT, 2), lambda i: (0, 0)),
            pl.BlockSpec((2, 1), lambda i: (0, 0)),
        ],
        out_specs=pl.BlockSpec((_PBLK,), lambda i: (i,)),
        out_shape=jax.ShapeDtypeStruct((_NPAD,), jnp.float32),
    )(features, W_clf, b_clf.reshape(2, 1))


_GB = 16
_GR = _GB * _K
_NG = _RPW // _GB


def _sc_body(feat_hbm, p_hbm, nodes_hbm, ncatf_hbm, self_hbm, agg_hbm,
             neighf_v, pnf_v, nodes3_v, pc_v, snodes_v, sel_v,
             rows0, rows1, agg0, agg1,
             semb, semg0, semg1, semw0, semw1):
    wid = lax.axis_index("s") * 2 + lax.axis_index("c")
    base = wid * _RPW

    cds = [pltpu.async_copy(
        ncatf_hbm.at[pl.ds(base * _DEG, _RPW * _DEG)], neighf_v, semb)]
    for ci in range(_RPW // _CH):
        cds.append(pltpu.async_copy(
            nodes_hbm.at[pl.ds(lax.rem(base + ci * _CH, _B), _CH)],
            nodes3_v.at[pl.ds(ci * _CH, _CH)], semb))
    cds.append(pltpu.async_copy(
        nodes_hbm.at[pl.ds(wid * _SPW, _SPW)], snodes_v, semb))
    for cd in cds:
        cd.wait()

    selmask = lax.iota(jnp.int32, 16) < _K

    def sel_body(b, c):
        ids = neighf_v[pl.ds(b * _DEG, _DEG)]
        pn = pnf_v[pl.ds(b * _DEG, _DEG)]
        pc = plsc.load_gather(pc_v, [jnp.full((16,), b, dtype=jnp.int32)])
        dist = jnp.abs(pn - pc)
        _, sids = plsc.sort_key_val(dist, ids)
        plsc.store_compressed(sel_v.at[pl.ds(b * _K, 16)], sids, mask=selmask)
        return c

    pc_cds = [pltpu.async_copy(
        p_hbm.at[nodes3_v.at[pl.ds(i * 128, 128)]],
        pc_v.at[pl.ds(i * 128, 128)], semb) for i in range(_RPW // 128)]

    n_grp = _RPW * _DEG // 128
    n_q = n_grp // 4

    def fire_quarter(q):
        return [pltpu.async_copy(
            p_hbm.at[neighf_v.at[pl.ds((q * n_q + i) * 128, 128)]],
            pnf_v.at[pl.ds((q * n_q + i) * 128, 128)], semb)
            for i in range(n_q)]

    pend = fire_quarter(0)
    for q in range(4):
        nxt = fire_quarter(q + 1) if q < 3 else []
        for cd in pend:
            cd.wait()
        if q == 0:
            for cd in pc_cds:
                cd.wait()
        lax.fori_loop(q * (_RPW // 4), (q + 1) * (_RPW // 4), sel_body, 0)
        pend = nxt

    def fire(g, buf, sem):
        gg = lax.rem(g, _NG)
        pltpu.async_copy(feat_hbm.at[sel_v.at[pl.ds(gg * _GR, _GR)]], buf,
                         sem)

    def gwait(buf, sem):
        pltpu.make_async_copy(
            feat_hbm.at[sel_v.at[pl.ds(0, _GR)]], buf, sem).wait()

    def wwait(agg, sem):
        pltpu.make_async_copy(agg, agg_hbm.at[pl.ds(0, _GB), :], sem).wait()

    def reduce_into(rows, agg):
        def j_body(j, c):
            for cc in range(_FEAT // 16):
                acc = rows[j * _K, pl.ds(cc * 16, 16)]
                for k in range(1, _K):
                    acc = acc + rows[j * _K + k, pl.ds(cc * 16, 16)]
                agg[j, pl.ds(cc * 16, 16)] = acc
            return c

        lax.fori_loop(0, _GB, j_body, 0)

    fire(0, rows0, semg0)

    def pipe_body(i, c):
        g0 = 2 * i
        g1 = g0 + 1
        fire(g1, rows1, semg1)
        gwait(rows0, semg0)

        @pl.when(i > 0)
        def _():
            wwait(agg0, semw0)

        reduce_into(rows0, agg0)
        pltpu.async_copy(agg0, agg_hbm.at[pl.ds(base + g0 * _GB, _GB), :],
                         semw0)
        fire(g0 + 2, rows0, semg0)
        gwait(rows1, semg1)

        @pl.when(i > 0)
        def _():
            wwait(agg1, semw1)

        reduce_into(rows1, agg1)
        pltpu.async_copy(agg1, agg_hbm.at[pl.ds(base + g1 * _GB, _GB), :],
                         semw1)
        return c

    lax.fori_loop(0, _NG // 2, pipe_body, 0)
    gwait(rows0, semg0)
    wwait(agg0, semw0)
    wwait(agg1, semw1)

    fire0 = pltpu.async_copy(
        feat_hbm.at[snodes_v.at[pl.ds(0, _GR)]], rows0, semg0)
    fire1 = pltpu.async_copy(
        feat_hbm.at[snodes_v.at[pl.ds(_GR, _GR)]], rows1, semg1)
    fire0.wait()
    pltpu.sync_copy(rows0, self_hbm.at[pl.ds(wid * _SPW, _GR), :])
    fire1.wait()
    pltpu.sync_copy(rows1, self_hbm.at[pl.ds(wid * _SPW + _GR, _GR), :])


def _sc_gather(features, p_all, nodes, ncatf):
    mesh = plsc.VectorSubcoreMesh(core_axis_name="c", subcore_axis_name="s")
    call = pl.kernel(
        _sc_body,
        out_type=(
            jax.ShapeDtypeStruct((_B, _FEAT), jnp.float32),
            jax.ShapeDtypeStruct((3 * _B, _FEAT), jnp.float32),
        ),
        mesh=mesh,
        compiler_params=pltpu.CompilerParams(needs_layout_passes=False),
        scratch_types=[
            pltpu.VMEM((_RPW * _DEG,), jnp.int32),
            pltpu.VMEM((_RPW * _DEG,), jnp.float32),
            pltpu.VMEM((_RPW,), jnp.int32),
            pltpu.VMEM((_RPW,), jnp.float32),
            pltpu.VMEM((_SPW,), jnp.int32),
            pltpu.VMEM((_RPW * _K + 16,), jnp.int32),
            pltpu.VMEM((_GR, _FEAT), jnp.float32),
            pltpu.VMEM((_GR, _FEAT), jnp.float32),
            pltpu.VMEM((_GB, _FEAT), jnp.float32),
            pltpu.VMEM((_GB, _FEAT), jnp.float32),
            pltpu.SemaphoreType.DMA,
            pltpu.SemaphoreType.DMA,
            pltpu.SemaphoreType.DMA,
            pltpu.SemaphoreType.DMA,
            pltpu.SemaphoreType.DMA,
        ],
    )
    return call(features, p_all, nodes, ncatf)


def _c_body(self_ref, a1, a2, a3, wr1, wr2, wr3, ws, w1, w2, w3, o_ref):
    acc = lax.dot_general(ws[...], self_ref[...], (((0,), (1,)), ((), ())),
                          preferred_element_type=jnp.float32)
    for a, wr, wc in ((a1, wr1, w1), (a2, wr2, w2), (a3, wr3, w3)):
        h = jnp.maximum(
            jnp.dot(a[...], wr[...], preferred_element_type=jnp.float32), 0.0)
        acc = acc + lax.dot_general(wc[...], h, (((0,), (1,)), ((), ())),
                                    preferred_element_type=jnp.float32)
    o_ref[...] = jnp.maximum(acc, 0.0)


def _combine(self_feats, aggsum, wr1, wr2, wr3, ws, w1, w2, w3):
    bk = 1024
    grid = _B // bk
    wspec = pl.BlockSpec((_FEAT, _EMB), lambda i: (0, 0))
    especk = pl.BlockSpec((_EMB, _EMB), lambda i: (0, 0))
    return pl.pallas_call(
        _c_body,
        grid=(grid,),
        in_specs=[
            pl.BlockSpec((bk, _FEAT), lambda i: (i, 0)),
            pl.BlockSpec((bk, _FEAT), lambda i: (i, 0)),
            pl.BlockSpec((bk, _FEAT), lambda i: (i + grid, 0)),
            pl.BlockSpec((bk, _FEAT), lambda i: (i + 2 * grid, 0)),
            wspec, wspec, wspec, wspec,
            especk, especk, especk,
        ],
        out_specs=pl.BlockSpec((_EMB, bk), lambda i: (0, i)),
        out_shape=jax.ShapeDtypeStruct((_EMB, _B), jnp.float32),
    )(self_feats, aggsum, aggsum, aggsum, wr1, wr2, wr3, ws, w1, w2, w3)


def kernel(features, pe_features, W_clf, b_clf, W_r1, W_r2, W_r3, weight,
           nodes, labels, neigh_r1, neigh_r2, neigh_r3, train_pos):
    p_all = _compute_p(features, W_clf, b_clf)
    ncatf = jnp.concatenate([neigh_r1, neigh_r2, neigh_r3], axis=0).reshape(-1)
    self_feats, aggsum = _sc_gather(features, p_all, nodes, ncatf)
    scale = jnp.float32(1.0 / _K)
    out = _combine(
        self_feats, aggsum,
        W_r1 * scale, W_r2 * scale, W_r3 * scale,
        weight[0:_FEAT],
        weight[_FEAT:_FEAT + _EMB],
        weight[_FEAT + _EMB:_FEAT + 2 * _EMB],
        weight[_FEAT + 2 * _EMB:_FEAT + 3 * _EMB],
    )
    return out


# speedup vs baseline: 5.9781x; 1.1960x over previous
import functools

import jax
import jax.numpy as jnp
from jax import lax
from jax.experimental import pallas as pl
from jax.experimental.pallas import tpu as pltpu
from jax.experimental.pallas import tpu_sc as plsc

_N, _FEAT, _EMB, _B, _DEG = 100000, 128, 64, 8192, 16
_PBLK = 4096
_NPAD = 25 * _PBLK
_NW = 32
_RPW = (3 * _B) // _NW
_CH = 64
_SPW = _B // _NW
_K = 8


def _p_body(x_ref, w_ref, b_ref, o_ref):
    sT = lax.dot_general(w_ref[...], x_ref[...], (((0,), (1,)), ((), ())),
                         preferred_element_type=jnp.float32) + b_ref[...]
    s0 = sT[0, :]
    s1 = sT[1, :]
    m = jnp.maximum(s0, s1)
    e0 = jnp.exp(s0 - m)
    e1 = jnp.exp(s1 - m)
    o_ref[...] = e1 / (e0 + e1)


def _compute_p(features, W_clf, b_clf):
    grid = _NPAD // _PBLK
    return pl.pallas_call(
        _p_body,
        grid=(grid,),
        in_specs=[
            pl.BlockSpec((_PBLK, _FEAT), lambda i: (i, 0)),
            pl.BlockSpec((_FEAT, 2), lambda i: (0, 0)),
            pl.BlockSpec((2, 1), lambda i: (0, 0)),
        ],
        out_specs=pl.BlockSpec((_PBLK,), lambda i: (i,)),
        out_shape=jax.ShapeDtypeStruct((_NPAD,), jnp.float32),
    )(features, W_clf, b_clf.reshape(2, 1))


_GB = 16
_GR = _GB * _K
_NG = _RPW // _GB


def _sc_body(feat_hbm, p_hbm, nodes_hbm, ncatf_hbm, self_hbm, agg_hbm,
             neighf_v, pnf_v, nodes3_v, pc_v, snodes_v, sel_v,
             rows0, rows1, rows2, agg0, agg1, agg2,
             semb, semg0, semg1, semg2, semw0, semw1, semw2):
    wid = lax.axis_index("s") * 2 + lax.axis_index("c")
    base = wid * _RPW

    cds = [pltpu.async_copy(
        ncatf_hbm.at[pl.ds(base * _DEG, _RPW * _DEG)], neighf_v, semb)]
    for ci in range(_RPW // _CH):
        cds.append(pltpu.async_copy(
            nodes_hbm.at[pl.ds(lax.rem(base + ci * _CH, _B), _CH)],
            nodes3_v.at[pl.ds(ci * _CH, _CH)], semb))
    cds.append(pltpu.async_copy(
        nodes_hbm.at[pl.ds(wid * _SPW, _SPW)], snodes_v, semb))
    for cd in cds:
        cd.wait()

    selmask = lax.iota(jnp.int32, 16) < _K

    def sel_body(b, c):
        ids = neighf_v[pl.ds(b * _DEG, _DEG)]
        pn = pnf_v[pl.ds(b * _DEG, _DEG)]
        pc = plsc.load_gather(pc_v, [jnp.full((16,), b, dtype=jnp.int32)])
        dist = jnp.abs(pn - pc)
        _, sids = plsc.sort_key_val(dist, ids)
        plsc.store_compressed(sel_v.at[pl.ds(b * _K, 16)], sids, mask=selmask)
        return c

    pc_cds = [pltpu.async_copy(
        p_hbm.at[nodes3_v.at[pl.ds(i * 128, 128)]],
        pc_v.at[pl.ds(i * 128, 128)], semb) for i in range(_RPW // 128)]

    n_grp = _RPW * _DEG // 128
    n_q = n_grp // 4

    def fire_quarter(q):
        return [pltpu.async_copy(
            p_hbm.at[neighf_v.at[pl.ds((q * n_q + i) * 128, 128)]],
            pnf_v.at[pl.ds((q * n_q + i) * 128, 128)], semb)
            for i in range(n_q)]

    pend = fire_quarter(0)
    for q in range(4):
        nxt = fire_quarter(q + 1) if q < 3 else []
        for cd in pend:
            cd.wait()
        if q == 0:
            for cd in pc_cds:
                cd.wait()
        lax.fori_loop(q * (_RPW // 4), (q + 1) * (_RPW // 4), sel_body, 0)
        pend = nxt

    def fire(g, buf, sem):
        gg = lax.rem(g, _NG)
        pltpu.async_copy(feat_hbm.at[sel_v.at[pl.ds(gg * _GR, _GR)]], buf,
                         sem)

    def gwait(buf, sem):
        pltpu.make_async_copy(
            feat_hbm.at[sel_v.at[pl.ds(0, _GR)]], buf, sem).wait()

    def wwait(agg, sem):
        pltpu.make_async_copy(agg, agg_hbm.at[pl.ds(0, _GB), :], sem).wait()

    def reduce_into(rows, agg):
        def j_body(j, c):
            for cc in range(_FEAT // 16):
                acc = rows[j * _K, pl.ds(cc * 16, 16)]
                for k in range(1, _K):
                    acc = acc + rows[j * _K + k, pl.ds(cc * 16, 16)]
                agg[j, pl.ds(cc * 16, 16)] = acc
            return c

        lax.fori_loop(0, _GB, j_body, 0)

    bufs = ((rows0, agg0, semg0, semw0),
            (rows1, agg1, semg1, semw1),
            (rows2, agg2, semg2, semw2))
    for d, (rb, _, sg, _) in enumerate(bufs):
        fire(d, rb, sg)

    def pipe_body(i, c):
        for d, (rb, ab, sg, sw) in enumerate(bufs):
            g = 3 * i + d
            gwait(rb, sg)

            @pl.when(i > 0)
            def _():
                wwait(ab, sw)

            reduce_into(rb, ab)
            pltpu.async_copy(ab, agg_hbm.at[pl.ds(base + g * _GB, _GB), :],
                             sw)
            fire(g + 3, rb, sg)
        return c

    lax.fori_loop(0, _NG // 3, pipe_body, 0)
    for _, (rb, ab, sg, sw) in enumerate(bufs):
        gwait(rb, sg)
        wwait(ab, sw)

    fire0 = pltpu.async_copy(
        feat_hbm.at[snodes_v.at[pl.ds(0, _GR)]], rows0, semg0)
    fire1 = pltpu.async_copy(
        feat_hbm.at[snodes_v.at[pl.ds(_GR, _GR)]], rows1, semg1)
    fire0.wait()
    pltpu.sync_copy(rows0, self_hbm.at[pl.ds(wid * _SPW, _GR), :])
    fire1.wait()
    pltpu.sync_copy(rows1, self_hbm.at[pl.ds(wid * _SPW + _GR, _GR), :])


def _sc_gather(features, p_all, nodes, ncatf):
    mesh = plsc.VectorSubcoreMesh(core_axis_name="c", subcore_axis_name="s")
    call = pl.kernel(
        _sc_body,
        out_type=(
            jax.ShapeDtypeStruct((_B, _FEAT), jnp.float32),
            jax.ShapeDtypeStruct((3 * _B, _FEAT), jnp.float32),
        ),
        mesh=mesh,
        compiler_params=pltpu.CompilerParams(needs_layout_passes=False),
        scratch_types=[
            pltpu.VMEM((_RPW * _DEG,), jnp.int32),
            pltpu.VMEM((_RPW * _DEG,), jnp.float32),
            pltpu.VMEM((_RPW,), jnp.int32),
            pltpu.VMEM((_RPW,), jnp.float32),
            pltpu.VMEM((_SPW,), jnp.int32),
            pltpu.VMEM((_RPW * _K + 16,), jnp.int32),
            pltpu.VMEM((_GR, _FEAT), jnp.float32),
            pltpu.VMEM((_GR, _FEAT), jnp.float32),
            pltpu.VMEM((_GR, _FEAT), jnp.float32),
            pltpu.VMEM((_GB, _FEAT), jnp.float32),
            pltpu.VMEM((_GB, _FEAT), jnp.float32),
            pltpu.VMEM((_GB, _FEAT), jnp.float32),
            pltpu.SemaphoreType.DMA,
            pltpu.SemaphoreType.DMA,
            pltpu.SemaphoreType.DMA,
            pltpu.SemaphoreType.DMA,
            pltpu.SemaphoreType.DMA,
            pltpu.SemaphoreType.DMA,
            pltpu.SemaphoreType.DMA,
        ],
    )
    return call(features, p_all, nodes, ncatf)


def _c_body(self_ref, a1, a2, a3, wr1, wr2, wr3, ws, w1, w2, w3, o_ref):
    acc = lax.dot_general(ws[...], self_ref[...], (((0,), (1,)), ((), ())),
                          preferred_element_type=jnp.float32)
    for a, wr, wc in ((a1, wr1, w1), (a2, wr2, w2), (a3, wr3, w3)):
        h = jnp.maximum(
            jnp.dot(a[...], wr[...], preferred_element_type=jnp.float32), 0.0)
        acc = acc + lax.dot_general(wc[...], h, (((0,), (1,)), ((), ())),
                                    preferred_element_type=jnp.float32)
    o_ref[...] = jnp.maximum(acc, 0.0)


def _combine(self_feats, aggsum, wr1, wr2, wr3, ws, w1, w2, w3):
    bk = 1024
    grid = _B // bk
    wspec = pl.BlockSpec((_FEAT, _EMB), lambda i: (0, 0))
    especk = pl.BlockSpec((_EMB, _EMB), lambda i: (0, 0))
    return pl.pallas_call(
        _c_body,
        grid=(grid,),
        in_specs=[
            pl.BlockSpec((bk, _FEAT), lambda i: (i, 0)),
            pl.BlockSpec((bk, _FEAT), lambda i: (i, 0)),
            pl.BlockSpec((bk, _FEAT), lambda i: (i + grid, 0)),
            pl.BlockSpec((bk, _FEAT), lambda i: (i + 2 * grid, 0)),
            wspec, wspec, wspec, wspec,
            especk, especk, especk,
        ],
        out_specs=pl.BlockSpec((_EMB, bk), lambda i: (0, i)),
        out_shape=jax.ShapeDtypeStruct((_EMB, _B), jnp.float32),
    )(self_feats, aggsum, aggsum, aggsum, wr1, wr2, wr3, ws, w1, w2, w3)


def kernel(features, pe_features, W_clf, b_clf, W_r1, W_r2, W_r3, weight,
           nodes, labels, neigh_r1, neigh_r2, neigh_r3, train_pos):
    p_all = _compute_p(features, W_clf, b_clf)
    ncatf = jnp.concatenate([neigh_r1, neigh_r2, neigh_r3], axis=0).reshape(-1)
    self_feats, aggsum = _sc_gather(features, p_all, nodes, ncatf)
    scale = jnp.float32(1.0 / _K)
    out = _combine(
        self_feats, aggsum,
        W_r1 * scale, W_r2 * scale, W_r3 * scale,
        weight[0:_FEAT],
        weight[_FEAT:_FEAT + _EMB],
        weight[_FEAT + _EMB:_FEAT + 2 * _EMB],
        weight[_FEAT + 2 * _EMB:_FEAT + 3 * _EMB],
    )
    return out


# speedup vs baseline: 6.0450x; 1.0112x over previous
import functools

import jax
import jax.numpy as jnp
from jax import lax
from jax.experimental import pallas as pl
from jax.experimental.pallas import tpu as pltpu
from jax.experimental.pallas import tpu_sc as plsc

_N, _FEAT, _EMB, _B, _DEG = 100000, 128, 64, 8192, 16
_PBLK = 4096
_NPAD = 25 * _PBLK
_NW = 32
_RPW = (3 * _B) // _NW
_CH = 64
_SPW = _B // _NW
_K = 8


def _p_body(x_ref, w_ref, b_ref, o_ref):
    sT = lax.dot_general(w_ref[...], x_ref[...], (((0,), (1,)), ((), ())),
                         preferred_element_type=jnp.float32) + b_ref[...]
    s0 = sT[0, :]
    s1 = sT[1, :]
    m = jnp.maximum(s0, s1)
    e0 = jnp.exp(s0 - m)
    e1 = jnp.exp(s1 - m)
    o_ref[...] = e1 / (e0 + e1)


def _compute_p(features, W_clf, b_clf):
    grid = _NPAD // _PBLK
    return pl.pallas_call(
        _p_body,
        grid=(grid,),
        in_specs=[
            pl.BlockSpec((_PBLK, _FEAT), lambda i: (i, 0)),
            pl.BlockSpec((_FEAT, 2), lambda i: (0, 0)),
            pl.BlockSpec((2, 1), lambda i: (0, 0)),
        ],
        out_specs=pl.BlockSpec((_PBLK,), lambda i: (i,)),
        out_shape=jax.ShapeDtypeStruct((_NPAD,), jnp.float32),
    )(features, W_clf, b_clf.reshape(2, 1))


_GB = 16
_GR = _GB * _K
_NG = _RPW // _GB


def _sc_body(feat_hbm, p_hbm, nodes_hbm, ncatf_hbm, self_hbm, agg_hbm,
             neighf_v, pnf_v, nodes3_v, pc_v, snodes_v, sel_v,
             rows0, rows1, rows2, agg0, agg1, agg2,
             semb, semg0, semg1, semg2, semw0, semw1, semw2):
    wid = lax.axis_index("s") * 2 + lax.axis_index("c")
    base = wid * _RPW

    cds = [pltpu.async_copy(
        ncatf_hbm.at[pl.ds(base * _DEG, _RPW * _DEG)], neighf_v, semb)]
    for ci in range(_RPW // _CH):
        cds.append(pltpu.async_copy(
            nodes_hbm.at[pl.ds(lax.rem(base + ci * _CH, _B), _CH)],
            nodes3_v.at[pl.ds(ci * _CH, _CH)], semb))
    cds.append(pltpu.async_copy(
        nodes_hbm.at[pl.ds(wid * _SPW, _SPW)], snodes_v, semb))
    for cd in cds:
        cd.wait()

    selmask = lax.iota(jnp.int32, 16) < _K

    def sel_body(b, c):
        ids = neighf_v[pl.ds(b * _DEG, _DEG)]
        pn = pnf_v[pl.ds(b * _DEG, _DEG)]
        pc = plsc.load_gather(pc_v, [jnp.full((16,), b, dtype=jnp.int32)])
        dist = jnp.abs(pn - pc)
        _, sids = plsc.sort_key_val(dist, ids)
        plsc.store_compressed(sel_v.at[pl.ds(b * _K, 16)], sids, mask=selmask)
        return c

    pc_cds = [pltpu.async_copy(
        p_hbm.at[nodes3_v.at[pl.ds(i * 128, 128)]],
        pc_v.at[pl.ds(i * 128, 128)], semb) for i in range(_RPW // 128)]

    n_grp = _RPW * _DEG // 128
    n_q = n_grp // 4

    def fire_quarter(q):
        return [pltpu.async_copy(
            p_hbm.at[neighf_v.at[pl.ds((q * n_q + i) * 128, 128)]],
            pnf_v.at[pl.ds((q * n_q + i) * 128, 128)], semb)
            for i in range(n_q)]

    def qsel(q):
        lax.fori_loop(q * (_RPW // 4), (q + 1) * (_RPW // 4), sel_body, 0)

    def fire(g, buf, sem):
        pltpu.async_copy(feat_hbm.at[sel_v.at[pl.ds(g * _GR, _GR)]], buf,
                         sem)

    def gwait(buf, sem):
        pltpu.make_async_copy(
            feat_hbm.at[sel_v.at[pl.ds(0, _GR)]], buf, sem).wait()

    def wwait(agg, sem):
        pltpu.make_async_copy(agg, agg_hbm.at[pl.ds(0, _GB), :], sem).wait()

    def reduce_into(rows, agg):
        def j_body(j, c):
            for cc in range(_FEAT // 16):
                acc = rows[j * _K, pl.ds(cc * 16, 16)]
                for k in range(1, _K):
                    acc = acc + rows[j * _K + k, pl.ds(cc * 16, 16)]
                agg[j, pl.ds(cc * 16, 16)] = acc
            return c

        lax.fori_loop(0, _GB, j_body, 0)

    bufs = ((rows0, agg0, semg0, semw0),
            (rows1, agg1, semg1, semw1),
            (rows2, agg2, semg2, semw2))

    pend = fire_quarter(0)
    for cd in pend:
        cd.wait()
    for cd in pc_cds:
        cd.wait()
    qsel(0)
    pendp = fire_quarter(1)
    nq = _NG // 4

    for q in range(4):
        for d, (rb, _, sg, _) in enumerate(bufs):
            fire(q * nq + d, rb, sg)
        if q < 3:
            for cd in pendp:
                cd.wait()
            qsel(q + 1)
            if q < 2:
                pendp = fire_quarter(q + 2)

        def p2_body(ii, c, q=q):
            for d, (rb, ab, sg, sw) in enumerate(bufs):
                i = ii * 3 + d
                g = q * nq + i
                gwait(rb, sg)
                if q == 0:
                    @pl.when(ii > 0)
                    def _():
                        wwait(ab, sw)
                else:
                    wwait(ab, sw)
                reduce_into(rb, ab)
                pltpu.async_copy(
                    ab, agg_hbm.at[pl.ds(base + g * _GB, _GB), :], sw)

                @pl.when(i < nq - 3)
                def _():
                    fire(g + 3, rb, sg)
            return c

        lax.fori_loop(0, nq // 3, p2_body, 0)

    for _, (rb, ab, sg, sw) in enumerate(bufs):
        wwait(ab, sw)

    fire0 = pltpu.async_copy(
        feat_hbm.at[snodes_v.at[pl.ds(0, _GR)]], rows0, semg0)
    fire1 = pltpu.async_copy(
        feat_hbm.at[snodes_v.at[pl.ds(_GR, _GR)]], rows1, semg1)
    fire0.wait()
    pltpu.sync_copy(rows0, self_hbm.at[pl.ds(wid * _SPW, _GR), :])
    fire1.wait()
    pltpu.sync_copy(rows1, self_hbm.at[pl.ds(wid * _SPW + _GR, _GR), :])


def _sc_gather(features, p_all, nodes, ncatf):
    mesh = plsc.VectorSubcoreMesh(core_axis_name="c", subcore_axis_name="s")
    call = pl.kernel(
        _sc_body,
        out_type=(
            jax.ShapeDtypeStruct((_B, _FEAT), jnp.float32),
            jax.ShapeDtypeStruct((3 * _B, _FEAT), jnp.float32),
        ),
        mesh=mesh,
        compiler_params=pltpu.CompilerParams(needs_layout_passes=False),
        scratch_types=[
            pltpu.VMEM((_RPW * _DEG,), jnp.int32),
            pltpu.VMEM((_RPW * _DEG,), jnp.float32),
            pltpu.VMEM((_RPW,), jnp.int32),
            pltpu.VMEM((_RPW,), jnp.float32),
            pltpu.VMEM((_SPW,), jnp.int32),
            pltpu.VMEM((_RPW * _K + 16,), jnp.int32),
            pltpu.VMEM((_GR, _FEAT), jnp.float32),
            pltpu.VMEM((_GR, _FEAT), jnp.float32),
            pltpu.VMEM((_GR, _FEAT), jnp.float32),
            pltpu.VMEM((_GB, _FEAT), jnp.float32),
            pltpu.VMEM((_GB, _FEAT), jnp.float32),
            pltpu.VMEM((_GB, _FEAT), jnp.float32),
            pltpu.SemaphoreType.DMA,
            pltpu.SemaphoreType.DMA,
            pltpu.SemaphoreType.DMA,
            pltpu.SemaphoreType.DMA,
            pltpu.SemaphoreType.DMA,
            pltpu.SemaphoreType.DMA,
            pltpu.SemaphoreType.DMA,
        ],
    )
    return call(features, p_all, nodes, ncatf)


def _c_body(self_ref, a1, a2, a3, wr1, wr2, wr3, ws, w1, w2, w3, o_ref):
    acc = lax.dot_general(ws[...], self_ref[...], (((0,), (1,)), ((), ())),
                          preferred_element_type=jnp.float32)
    for a, wr, wc in ((a1, wr1, w1), (a2, wr2, w2), (a3, wr3, w3)):
        h = jnp.maximum(
            jnp.dot(a[...], wr[...], preferred_element_type=jnp.float32), 0.0)
        acc = acc + lax.dot_general(wc[...], h, (((0,), (1,)), ((), ())),
                                    preferred_element_type=jnp.float32)
    o_ref[...] = jnp.maximum(acc, 0.0)


def _combine(self_feats, aggsum, wr1, wr2, wr3, ws, w1, w2, w3):
    bk = 1024
    grid = _B // bk
    wspec = pl.BlockSpec((_FEAT, _EMB), lambda i: (0, 0))
    especk = pl.BlockSpec((_EMB, _EMB), lambda i: (0, 0))
    return pl.pallas_call(
        _c_body,
        grid=(grid,),
        in_specs=[
            pl.BlockSpec((bk, _FEAT), lambda i: (i, 0)),
            pl.BlockSpec((bk, _FEAT), lambda i: (i, 0)),
            pl.BlockSpec((bk, _FEAT), lambda i: (i + grid, 0)),
            pl.BlockSpec((bk, _FEAT), lambda i: (i + 2 * grid, 0)),
            wspec, wspec, wspec, wspec,
            especk, especk, especk,
        ],
        out_specs=pl.BlockSpec((_EMB, bk), lambda i: (0, i)),
        out_shape=jax.ShapeDtypeStruct((_EMB, _B), jnp.float32),
    )(self_feats, aggsum, aggsum, aggsum, wr1, wr2, wr3, ws, w1, w2, w3)


def kernel(features, pe_features, W_clf, b_clf, W_r1, W_r2, W_r3, weight,
           nodes, labels, neigh_r1, neigh_r2, neigh_r3, train_pos):
    p_all = _compute_p(features, W_clf, b_clf)
    ncatf = jnp.concatenate([neigh_r1, neigh_r2, neigh_r3], axis=0).reshape(-1)
    self_feats, aggsum = _sc_gather(features, p_all, nodes, ncatf)
    scale = jnp.float32(1.0 / _K)
    out = _combine(
        self_feats, aggsum,
        W_r1 * scale, W_r2 * scale, W_r3 * scale,
        weight[0:_FEAT],
        weight[_FEAT:_FEAT + _EMB],
        weight[_FEAT + _EMB:_FEAT + 2 * _EMB],
        weight[_FEAT + 2 * _EMB:_FEAT + 3 * _EMB],
    )
    return out


# speedup vs baseline: 6.0994x; 1.0090x over previous
import functools

import jax
import jax.numpy as jnp
from jax import lax
from jax.experimental import pallas as pl
from jax.experimental.pallas import tpu as pltpu
from jax.experimental.pallas import tpu_sc as plsc

_N, _FEAT, _EMB, _B, _DEG = 100000, 128, 64, 8192, 16
_PBLK = 4096
_NPAD = 25 * _PBLK
_NW = 32
_RPW = (3 * _B) // _NW
_CH = 64
_SPW = _B // _NW
_K = 8


def _p_body(x_ref, w_ref, b_ref, o_ref):
    sT = lax.dot_general(w_ref[...], x_ref[...], (((0,), (1,)), ((), ())),
                         preferred_element_type=jnp.float32) + b_ref[...]
    s0 = sT[0, :]
    s1 = sT[1, :]
    m = jnp.maximum(s0, s1)
    e0 = jnp.exp(s0 - m)
    e1 = jnp.exp(s1 - m)
    o_ref[...] = e1 / (e0 + e1)


def _compute_p(features, W_clf, b_clf):
    grid = _NPAD // _PBLK
    return pl.pallas_call(
        _p_body,
        grid=(grid,),
        in_specs=[
            pl.BlockSpec((_PBLK, _FEAT), lambda i: (i, 0)),
            pl.BlockSpec((_FEAT, 2), lambda i: (0, 0)),
            pl.BlockSpec((2, 1), lambda i: (0, 0)),
        ],
        out_specs=pl.BlockSpec((_PBLK,), lambda i: (i,)),
        out_shape=jax.ShapeDtypeStruct((_NPAD,), jnp.float32),
    )(features, W_clf, b_clf.reshape(2, 1))


_GB = 16
_GR = _GB * _K
_NG = _RPW // _GB


def _sc_body(feat_hbm, p_hbm, nodes_hbm, ncatf_hbm, self_hbm, agg_hbm,
             neighf_v, pnf_v, nodes3_v, pc_v, snodes_v, sel_v,
             rows0, rows1, rows2, agg0, agg1, agg2,
             semb, semg0, semg1, semg2, semw0, semw1, semw2):
    wid = lax.axis_index("s") * 2 + lax.axis_index("c")
    base = wid * _RPW

    cds = [pltpu.async_copy(
        ncatf_hbm.at[pl.ds(base * _DEG, _RPW * _DEG)], neighf_v, semb)]
    for ci in range(_RPW // _CH):
        cds.append(pltpu.async_copy(
            nodes_hbm.at[pl.ds(lax.rem(base + ci * _CH, _B), _CH)],
            nodes3_v.at[pl.ds(ci * _CH, _CH)], semb))
    cds.append(pltpu.async_copy(
        nodes_hbm.at[pl.ds(wid * _SPW, _SPW)], snodes_v, semb))
    for cd in cds:
        cd.wait()

    selmask = lax.iota(jnp.int32, 16) < _K

    def sel_body(b, c):
        ids = neighf_v[pl.ds(b * _DEG, _DEG)]
        pn = pnf_v[pl.ds(b * _DEG, _DEG)]
        pc = plsc.load_gather(pc_v, [jnp.full((16,), b, dtype=jnp.int32)])
        dist = jnp.abs(pn - pc)
        _, sids = plsc.sort_key_val(dist, ids)
        plsc.store_compressed(sel_v.at[pl.ds(b * _K, 16)], sids, mask=selmask)
        return c

    pc_cds = [pltpu.async_copy(
        p_hbm.at[nodes3_v.at[pl.ds(i * 128, 128)]],
        pc_v.at[pl.ds(i * 128, 128)], semb) for i in range(_RPW // 128)]

    n_grp = _RPW * _DEG // 128
    n_q = n_grp // 4

    def fire_quarter(q):
        return [pltpu.async_copy(
            p_hbm.at[neighf_v.at[pl.ds((q * n_q + i) * 128, 128)]],
            pnf_v.at[pl.ds((q * n_q + i) * 128, 128)], semb)
            for i in range(n_q)]

    def qsel(q):
        lax.fori_loop(q * (_RPW // 4), (q + 1) * (_RPW // 4), sel_body, 0)

    def fire(g, buf, sem):
        pltpu.async_copy(feat_hbm.at[sel_v.at[pl.ds(g * _GR, _GR)]], buf,
                         sem)

    def gwait(buf, sem):
        pltpu.make_async_copy(
            feat_hbm.at[sel_v.at[pl.ds(0, _GR)]], buf, sem).wait()

    def wwait(agg, sem):
        pltpu.make_async_copy(agg, agg_hbm.at[pl.ds(0, _GB), :], sem).wait()

    def reduce_into(rows, agg):
        def j_body(j, c):
            for cc in range(_FEAT // 16):
                acc = rows[j * _K, pl.ds(cc * 16, 16)]
                for k in range(1, _K):
                    acc = acc + rows[j * _K + k, pl.ds(cc * 16, 16)]
                agg[j, pl.ds(cc * 16, 16)] = acc
            return c

        lax.fori_loop(0, _GB, j_body, 0)

    bufs = ((rows0, agg0, semg0, semw0),
            (rows1, agg1, semg1, semw1),
            (rows2, agg2, semg2, semw2))

    pend = fire_quarter(0)
    for cd in pend:
        cd.wait()
    for cd in pc_cds:
        cd.wait()
    qsel(0)
    pendp = fire_quarter(1)
    nq = _NG // 4

    for q in range(4):
        for d, (rb, _, sg, _) in enumerate(bufs):
            fire(q * nq + d, rb, sg)
        if q < 3:
            for cd in pendp:
                cd.wait()
            qsel(q + 1)
            if q < 2:
                pendp = fire_quarter(q + 2)

        def p2_body(ii, c, q=q):
            for d, (rb, ab, sg, sw) in enumerate(bufs):
                i = ii * 3 + d
                g = q * nq + i
                gwait(rb, sg)
                if q == 0:
                    @pl.when(ii > 0)
                    def _():
                        wwait(ab, sw)
                else:
                    wwait(ab, sw)
                reduce_into(rb, ab)
                pltpu.async_copy(
                    ab, agg_hbm.at[pl.ds(base + g * _GB, _GB), :], sw)

                @pl.when(i < nq - 3)
                def _():
                    fire(g + 3, rb, sg)
            return c

        lax.fori_loop(0, nq // 3, p2_body, 0)

    for _, (rb, ab, sg, sw) in enumerate(bufs):
        wwait(ab, sw)

    fire0 = pltpu.async_copy(
        feat_hbm.at[snodes_v.at[pl.ds(0, _GR)]], rows0, semg0)
    fire1 = pltpu.async_copy(
        feat_hbm.at[snodes_v.at[pl.ds(_GR, _GR)]], rows1, semg1)
    fire0.wait()
    pltpu.sync_copy(rows0, self_hbm.at[pl.ds(wid * _SPW, _GR), :])
    fire1.wait()
    pltpu.sync_copy(rows1, self_hbm.at[pl.ds(wid * _SPW + _GR, _GR), :])


def _sc_gather(features, p_all, nodes, ncatf):
    mesh = plsc.VectorSubcoreMesh(core_axis_name="c", subcore_axis_name="s")
    call = pl.kernel(
        _sc_body,
        out_type=(
            jax.ShapeDtypeStruct((_B, _FEAT), jnp.float32),
            jax.ShapeDtypeStruct((3 * _B, _FEAT), jnp.float32),
        ),
        mesh=mesh,
        compiler_params=pltpu.CompilerParams(needs_layout_passes=False),
        scratch_types=[
            pltpu.VMEM((_RPW * _DEG,), jnp.int32),
            pltpu.VMEM((_RPW * _DEG,), jnp.float32),
            pltpu.VMEM((_RPW,), jnp.int32),
            pltpu.VMEM((_RPW,), jnp.float32),
            pltpu.VMEM((_SPW,), jnp.int32),
            pltpu.VMEM((_RPW * _K + 16,), jnp.int32),
            pltpu.VMEM((_GR, _FEAT), jnp.float32),
            pltpu.VMEM((_GR, _FEAT), jnp.float32),
            pltpu.VMEM((_GR, _FEAT), jnp.float32),
            pltpu.VMEM((_GB, _FEAT), jnp.float32),
            pltpu.VMEM((_GB, _FEAT), jnp.float32),
            pltpu.VMEM((_GB, _FEAT), jnp.float32),
            pltpu.SemaphoreType.DMA,
            pltpu.SemaphoreType.DMA,
            pltpu.SemaphoreType.DMA,
            pltpu.SemaphoreType.DMA,
            pltpu.SemaphoreType.DMA,
            pltpu.SemaphoreType.DMA,
            pltpu.SemaphoreType.DMA,
        ],
    )
    return call(features, p_all, nodes, ncatf)


def _c_body(self_ref, a1, a2, a3, wr1, wr2, wr3, ws, w1, w2, w3, o_ref):
    acc = lax.dot_general(ws[...], self_ref[...], (((0,), (1,)), ((), ())),
                          preferred_element_type=jnp.float32)
    for a, wr, wc in ((a1, wr1, w1), (a2, wr2, w2), (a3, wr3, w3)):
        h = jnp.maximum(
            jnp.dot(a[...], wr[...], preferred_element_type=jnp.float32), 0.0)
        acc = acc + lax.dot_general(wc[...], h, (((0,), (1,)), ((), ())),
                                    preferred_element_type=jnp.float32)
    o_ref[...] = jnp.maximum(acc, 0.0)


def _combine(self_feats, aggsum, wr1, wr2, wr3, ws, w1, w2, w3):
    bk = 2048
    grid = _B // bk
    wspec = pl.BlockSpec((_FEAT, _EMB), lambda i: (0, 0))
    especk = pl.BlockSpec((_EMB, _EMB), lambda i: (0, 0))
    return pl.pallas_call(
        _c_body,
        grid=(grid,),
        in_specs=[
            pl.BlockSpec((bk, _FEAT), lambda i: (i, 0)),
            pl.BlockSpec((bk, _FEAT), lambda i: (i, 0)),
            pl.BlockSpec((bk, _FEAT), lambda i: (i + grid, 0)),
            pl.BlockSpec((bk, _FEAT), lambda i: (i + 2 * grid, 0)),
            wspec, wspec, wspec, wspec,
            especk, especk, especk,
        ],
        out_specs=pl.BlockSpec((_EMB, bk), lambda i: (0, i)),
        out_shape=jax.ShapeDtypeStruct((_EMB, _B), jnp.float32),
    )(self_feats, aggsum, aggsum, aggsum, wr1, wr2, wr3, ws, w1, w2, w3)


def kernel(features, pe_features, W_clf, b_clf, W_r1, W_r2, W_r3, weight,
           nodes, labels, neigh_r1, neigh_r2, neigh_r3, train_pos):
    p_all = _compute_p(features, W_clf, b_clf)
    ncatf = jnp.concatenate([neigh_r1, neigh_r2, neigh_r3], axis=0).reshape(-1)
    self_feats, aggsum = _sc_gather(features, p_all, nodes, ncatf)
    scale = jnp.float32(1.0 / _K)
    out = _combine(
        self_feats, aggsum,
        W_r1 * scale, W_r2 * scale, W_r3 * scale,
        weight[0:_FEAT],
        weight[_FEAT:_FEAT + _EMB],
        weight[_FEAT + _EMB:_FEAT + 2 * _EMB],
        weight[_FEAT + 2 * _EMB:_FEAT + 3 * _EMB],
    )
    return out
